# baseline (speedup 1.0000x reference)
import functools

import jax
import jax.numpy as jnp
from jax import lax
from jax.experimental import pallas as pl
from jax.experimental.pallas import tpu as pltpu
from jax.experimental.pallas import tpu_sc as plsc

N = 10000
E = 320000
D = 128

NC = 2
NS = 16
NW = NC * NS
CHUNK = 128
PH = 16
NPHASE = 5
CPW = PH * NPHASE
E_PAD = NW * CPW * CHUNK
PAD = E_PAD - E
NP = 10112
NDUM = NP - N
RPT = NP // NS
WLAST = N - (NS - 1) * RPT
HR = 80
L = 16


def _sc_body(src_hbm, dst_hbm, x_hbm, zacc_hbm, zflat_hbm,
             part_hbm, cntp_hbm,
             src_v, dst_v, rows_v, hist_v, acc_sh, sem_a, sem_b, sem_s):
    cid = lax.axis_index("c")
    sid = lax.axis_index("s")
    wid = cid * NS + sid
    roff = pl.multiple_of(sid * RPT, 8)

    pltpu.sync_copy(zacc_hbm.at[pl.ds(roff, RPT)],
                    acc_sh.at[pl.ds(roff, RPT)])
    pltpu.sync_copy(zflat_hbm, hist_v)
    plsc.subcore_barrier()

    def consume(j, b):
        sc = pltpu.async_copy(rows_v.at[b], acc_sh.at[dst_v.at[j]], sem_s,
                              add=True)
        for k in range(CHUNK // L):
            dstv = dst_v[j, pl.ds(k * L, L)]
            vals = jnp.full((L,), 1.0, jnp.float32)
            plsc.addupdate_scatter(hist_v, [dstv], vals)
        sc.wait()

    for ph in range(NPHASE):
        pltpu.sync_copy(src_hbm.at[wid, pl.ds(ph * PH, PH)], src_v)
        pltpu.sync_copy(dst_hbm.at[wid, pl.ds(ph * PH, PH)], dst_v)

        pltpu.make_async_copy(x_hbm.at[src_v.at[0]], rows_v.at[0],
                              sem_a).start()

        def pair(i, carry):
            j0 = 2 * i
            pltpu.make_async_copy(x_hbm.at[src_v.at[j0 + 1]], rows_v.at[1],
                                  sem_b).start()
            pltpu.make_async_copy(x_hbm.at[src_v.at[j0]], rows_v.at[0],
                                  sem_a).wait()
            consume(j0, 0)

            @pl.when(j0 + 2 < PH)
            def _prefetch():
                pltpu.make_async_copy(x_hbm.at[src_v.at[j0 + 2]],
                                      rows_v.at[0], sem_a).start()

            pltpu.make_async_copy(x_hbm.at[src_v.at[j0 + 1]], rows_v.at[1],
                                  sem_b).wait()
            consume(j0 + 1, 1)
            return carry

        lax.fori_loop(0, PH // 2, pair, 0)

    plsc.subcore_barrier()

    @pl.when(sid < NS - 1)
    def _full():
        pltpu.sync_copy(acc_sh.at[pl.ds(roff, RPT)],
                        part_hbm.at[cid, pl.ds(roff, RPT)])

    @pl.when(sid == NS - 1)
    def _last():
        pltpu.sync_copy(acc_sh.at[pl.ds((NS - 1) * RPT, WLAST)],
                        part_hbm.at[cid, pl.ds((NS - 1) * RPT, WLAST)])

    pltpu.sync_copy(hist_v, cntp_hbm.at[wid, 0])


_sc_call = functools.partial(
    pl.kernel,
    out_type=(
        jax.ShapeDtypeStruct((NC, N, D), jnp.float32),
        jax.ShapeDtypeStruct((NW, 1, HR * 128), jnp.float32),
    ),
    compiler_params=pltpu.CompilerParams(needs_layout_passes=False),
    mesh=plsc.VectorSubcoreMesh(core_axis_name="c", subcore_axis_name="s"),
    scratch_types=[
        pltpu.VMEM((PH, CHUNK), jnp.int32),
        pltpu.VMEM((PH, CHUNK), jnp.int32),
        pltpu.VMEM((2, CHUNK, D), jnp.float32),
        pltpu.VMEM((HR * 128,), jnp.float32),
        pltpu.VMEM_SHARED((NP, D), jnp.float32),
        pltpu.SemaphoreType.DMA,
        pltpu.SemaphoreType.DMA,
        pltpu.SemaphoreType.DMA,
    ],
)(_sc_body)


BLK = 1000


def _tc_body(p_ref, c_ref, x_ref, wl_ref, wr_ref, b_ref, o_ref, cc_ref):
    i = pl.program_id(0)

    @pl.when(i == 0)
    def _counts():
        ones_w = jnp.ones((NW, 1), jnp.float32)
        cc_ref[...] = lax.dot_general(
            c_ref[...], ones_w, (((0,), (0,)), ((), ())),
            preferred_element_type=jnp.float32)

    s = p_ref[0] + p_ref[1]
    cnt = jnp.maximum(cc_ref[pl.ds(i * BLK, BLK), :], 1.0)
    agg = s / cnt
    o_ref[...] = (
        jnp.dot(agg, wl_ref[...], preferred_element_type=jnp.float32)
        + jnp.dot(x_ref[...], wr_ref[...], preferred_element_type=jnp.float32)
        + b_ref[...]
    )


_tc_call = pl.pallas_call(
    _tc_body,
    out_shape=jax.ShapeDtypeStruct((N, D), jnp.float32),
    grid=(N // BLK,),
    in_specs=[
        pl.BlockSpec((NC, BLK, D), lambda i: (0, i, 0)),
        pl.BlockSpec((NW, HR * 128), lambda i: (0, 0)),
        pl.BlockSpec((BLK, D), lambda i: (i, 0)),
        pl.BlockSpec((D, D), lambda i: (0, 0)),
        pl.BlockSpec((D, D), lambda i: (0, 0)),
        pl.BlockSpec((1, D), lambda i: (0, 0)),
    ],
    out_specs=pl.BlockSpec((BLK, D), lambda i: (i, 0)),
    scratch_shapes=[pltpu.VMEM((HR * 128, 1), jnp.float32)],
)


def kernel(x, edge_index, W_l, b_l, W_r):
    src = edge_index[0]
    dst = edge_index[1]
    pad_src = (jnp.arange(PAD, dtype=jnp.int32) * 37) % N
    pad_dst = N + (jnp.arange(PAD, dtype=jnp.int32) % NDUM)
    src_p = jnp.concatenate([src, pad_src]).reshape(NW, CPW, CHUNK)
    dst_p = jnp.concatenate([dst, pad_dst]).reshape(NW, CPW, CHUNK)

    zacc = jnp.zeros((NP, D), jnp.float32)
    zflat = jnp.zeros((HR * 128,), jnp.float32)

    part, cntp = _sc_call(src_p, dst_p, x, zacc, zflat)
    cnt32 = cntp.reshape(NW, HR * 128)
    return _tc_call(part, cnt32, x, W_l.T, W_r.T, b_l.reshape(1, D))


# speedup vs baseline: 1.0687x; 1.0687x over previous
import functools

import jax
import jax.numpy as jnp
from jax import lax
from jax.experimental import pallas as pl
from jax.experimental.pallas import tpu as pltpu
from jax.experimental.pallas import tpu_sc as plsc

N = 10000
E = 320000
D = 128

NC = 2
NS = 16
NW = NC * NS
CHUNK = 128
PH = 16
NPHASE = 5
CPW = PH * NPHASE
E_PAD = NW * CPW * CHUNK
PAD = E_PAD - E
NP = 10112
NDUM = NP - N
RPT = NP // NS
WLAST = N - (NS - 1) * RPT
HR = 80
L = 16


def _sc_body(src_hbm, dst_hbm, x_hbm, zacc_hbm, zflat_hbm,
             part_hbm, cntp_hbm,
             src_v, dst_v, rows_v, hist_v, acc_sh, sem_a, sem_b, sem_s):
    cid = lax.axis_index("c")
    sid = lax.axis_index("s")
    wid = cid * NS + sid
    roff = pl.multiple_of(sid * RPT, 8)

    zc = pltpu.async_copy(zacc_hbm, acc_sh.at[pl.ds(roff, RPT)], sem_s)
    pltpu.sync_copy(zflat_hbm, hist_v)
    pltpu.sync_copy(src_hbm.at[wid, pl.ds(0, PH)], src_v)
    pltpu.sync_copy(dst_hbm.at[wid, pl.ds(0, PH)], dst_v)
    pltpu.make_async_copy(x_hbm.at[src_v.at[0]], rows_v.at[0],
                          sem_a).start()
    zc.wait()
    plsc.subcore_barrier()

    def consume(j, b):
        sc = pltpu.async_copy(rows_v.at[b], acc_sh.at[dst_v.at[j]], sem_s,
                              add=True)
        for k in range(CHUNK // L):
            dstv = dst_v[j, pl.ds(k * L, L)]
            vals = jnp.full((L,), 1.0, jnp.float32)
            plsc.addupdate_scatter(hist_v, [dstv], vals)
        sc.wait()

    for ph in range(NPHASE):
        if ph > 0:
            pltpu.sync_copy(src_hbm.at[wid, pl.ds(ph * PH, PH)], src_v)
            pltpu.sync_copy(dst_hbm.at[wid, pl.ds(ph * PH, PH)], dst_v)
            pltpu.make_async_copy(x_hbm.at[src_v.at[0]], rows_v.at[0],
                                  sem_a).start()

        def pair(i, carry):
            j0 = 2 * i
            pltpu.make_async_copy(x_hbm.at[src_v.at[j0 + 1]], rows_v.at[1],
                                  sem_b).start()
            pltpu.make_async_copy(x_hbm.at[src_v.at[j0]], rows_v.at[0],
                                  sem_a).wait()
            consume(j0, 0)

            @pl.when(j0 + 2 < PH)
            def _prefetch():
                pltpu.make_async_copy(x_hbm.at[src_v.at[j0 + 2]],
                                      rows_v.at[0], sem_a).start()

            pltpu.make_async_copy(x_hbm.at[src_v.at[j0 + 1]], rows_v.at[1],
                                  sem_b).wait()
            consume(j0 + 1, 1)
            return carry

        lax.fori_loop(0, PH // 2, pair, 0)

    plsc.subcore_barrier()

    @pl.when(sid < NS - 1)
    def _full():
        pltpu.sync_copy(acc_sh.at[pl.ds(roff, RPT)],
                        part_hbm.at[cid, pl.ds(roff, RPT)])

    @pl.when(sid == NS - 1)
    def _last():
        pltpu.sync_copy(acc_sh.at[pl.ds((NS - 1) * RPT, WLAST)],
                        part_hbm.at[cid, pl.ds((NS - 1) * RPT, WLAST)])

    pltpu.sync_copy(hist_v, cntp_hbm.at[wid, 0])


_sc_call = functools.partial(
    pl.kernel,
    out_type=(
        jax.ShapeDtypeStruct((NC, N, D), jnp.float32),
        jax.ShapeDtypeStruct((NW, 1, HR * 128), jnp.float32),
    ),
    compiler_params=pltpu.CompilerParams(needs_layout_passes=False),
    mesh=plsc.VectorSubcoreMesh(core_axis_name="c", subcore_axis_name="s"),
    scratch_types=[
        pltpu.VMEM((PH, CHUNK), jnp.int32),
        pltpu.VMEM((PH, CHUNK), jnp.int32),
        pltpu.VMEM((2, CHUNK, D), jnp.float32),
        pltpu.VMEM((HR * 128,), jnp.float32),
        pltpu.VMEM_SHARED((NP, D), jnp.float32),
        pltpu.SemaphoreType.DMA,
        pltpu.SemaphoreType.DMA,
        pltpu.SemaphoreType.DMA,
    ],
)(_sc_body)


def _tc_body(p_ref, c_ref, x_ref, wl_ref, wr_ref, b_ref, o_ref):
    s = p_ref[0] + p_ref[1]
    ones_w = jnp.ones((NW, 1), jnp.float32)
    cc = lax.dot_general(c_ref[...], ones_w, (((0,), (0,)), ((), ())),
                         preferred_element_type=jnp.float32)
    cnt = jnp.maximum(cc[:N], 1.0)
    agg = s / cnt
    o_ref[...] = (
        jnp.dot(agg, wl_ref[...], preferred_element_type=jnp.float32)
        + jnp.dot(x_ref[...], wr_ref[...], preferred_element_type=jnp.float32)
        + b_ref[...]
    )


_tc_call = pl.pallas_call(
    _tc_body,
    out_shape=jax.ShapeDtypeStruct((N, D), jnp.float32),
    grid=(1,),
    in_specs=[
        pl.BlockSpec((NC, N, D), lambda i: (0, 0, 0)),
        pl.BlockSpec((NW, HR * 128), lambda i: (0, 0)),
        pl.BlockSpec((N, D), lambda i: (0, 0)),
        pl.BlockSpec((D, D), lambda i: (0, 0)),
        pl.BlockSpec((D, D), lambda i: (0, 0)),
        pl.BlockSpec((1, D), lambda i: (0, 0)),
    ],
    out_specs=pl.BlockSpec((N, D), lambda i: (0, 0)),
)


def kernel(x, edge_index, W_l, b_l, W_r):
    src = edge_index[0]
    dst = edge_index[1]
    pad_src = (jnp.arange(PAD, dtype=jnp.int32) * 37) % N
    pad_dst = N + (jnp.arange(PAD, dtype=jnp.int32) % NDUM)
    ei_pad = jnp.concatenate(
        [edge_index, jnp.stack([pad_src, pad_dst])], axis=1)
    src_p = ei_pad[0].reshape(NW, CPW, CHUNK)
    dst_p = ei_pad[1].reshape(NW, CPW, CHUNK)

    zacc = jnp.zeros((RPT, D), jnp.float32)
    zflat = jnp.zeros((HR * 128,), jnp.float32)

    part, cntp = _sc_call(src_p, dst_p, x, zacc, zflat)
    cnt32 = cntp.reshape(NW, HR * 128)
    return _tc_call(part, cnt32, x, W_l.T, W_r.T, b_l.reshape(1, D))
